# baseline (speedup 1.0000x reference)
import functools

import jax
import jax.numpy as jnp
from jax import lax
from jax.experimental import pallas as pl
from jax.experimental.pallas import tpu as pltpu
from jax.experimental.pallas import tpu_sc as plsc

D = 32
LANES = 16
NC, NS = 2, 16
NW = NC * NS

BATCH = 4096
LEN_CONTEXT = 200
BT = BATCH // NW
DT = D // 8
TILE = 8 * BT

_mesh = plsc.VectorSubcoreMesh(core_axis_name="c", subcore_axis_name="s")


@functools.partial(
    pl.kernel,
    mesh=_mesh,
    compiler_params=pltpu.CompilerParams(
        use_tc_tiling_on_sc=False, needs_layout_passes=False),
    out_type=jax.ShapeDtypeStruct((2, LEN_CONTEXT, DT, NW * TILE), jnp.float32),
    scratch_types=[
        pltpu.VMEM((2, BT), jnp.int32),
        pltpu.VMEM((2, BT), jnp.int32),
        pltpu.VMEM((2, BT), jnp.int32),
        pltpu.VMEM((2, BT, D), jnp.float32),
        pltpu.VMEM((2, BT, D), jnp.float32),
        pltpu.VMEM((2, BT, D), jnp.float32),
        pltpu.VMEM((2, DT * TILE), jnp.float32),
        pltpu.VMEM((2, DT * TILE), jnp.float32),
        pltpu.SemaphoreType.DMA,
        pltpu.SemaphoreType.DMA,
        pltpu.SemaphoreType.DMA,
        pltpu.SemaphoreType.DMA,
    ],
)
def _embed_sc(state_hbm, action_hbm, lp_hbm, ws_hbm, wa_hbm, wp_hbm,
              out_hbm, idx_s, idx_a, idx_p, rows_s, rows_a, rows_p,
              out_s, out_a, gsem0, gsem1, osem0, osem1):
    wid = lax.axis_index("s") * NC + lax.axis_index("c")
    b0 = wid * BT
    biota = lax.iota(jnp.int32, LANES)

    gsems = (gsem0, gsem1)
    osems = (osem0, osem1)

    def fire_gathers(l, slot):
        sem = gsems[slot]
        pltpu.sync_copy(state_hbm.at[l, pl.ds(b0, BT)], idx_s.at[slot])
        pltpu.sync_copy(action_hbm.at[l, pl.ds(b0, BT)], idx_a.at[slot])
        pltpu.sync_copy(lp_hbm.at[l, pl.ds(b0, BT)], idx_p.at[slot])
        pltpu.async_copy(ws_hbm.at[idx_s.at[slot]], rows_s.at[slot], sem)
        pltpu.async_copy(wa_hbm.at[idx_a.at[slot]], rows_a.at[slot], sem)
        pltpu.async_copy(wp_hbm.at[idx_p.at[slot]], rows_p.at[slot], sem)

    def drain_gathers(slot):
        sem = gsems[slot]
        pltpu.make_async_copy(ws_hbm.at[idx_s.at[slot]], rows_s.at[slot], sem).wait()
        pltpu.make_async_copy(wa_hbm.at[idx_a.at[slot]], rows_a.at[slot], sem).wait()
        pltpu.make_async_copy(wp_hbm.at[idx_p.at[slot]], rows_p.at[slot], sem).wait()

    def compute(slot):
        rs, ra, rp = rows_s.at[slot], rows_a.at[slot], rows_p.at[slot]
        os_, oa = out_s.at[slot], out_a.at[slot]

        def body(t, _):
            bg = t >> 4
            d = (t & 15) * 2
            bvec = biota + bg * LANES
            dst = d * BT + bg * LANES
            for k in range(2):
                dvec = jnp.broadcast_to(d + k, (LANES,))
                p1 = plsc.load_gather(rp, [bvec, dvec]) + 1.0
                for rows, obuf in ((rs, os_), (ra, oa)):
                    x = plsc.load_gather(rows, [bvec, dvec])
                    e = jnp.exp(x + x)
                    r = 1.0 / (e + 1.0)
                    obuf[pl.ds(dst + k * BT, LANES)] = p1 - (r + r)
            return 0

        lax.fori_loop(0, BT, body, 0)

    def fire_out(l, slot):
        sem = osems[slot]
        for st, obuf in ((0, out_s), (1, out_a)):
            for dt in range(DT):
                pltpu.async_copy(
                    obuf.at[slot, pl.ds(dt * TILE, TILE)],
                    out_hbm.at[st, l, dt, pl.ds(wid * TILE, TILE)],
                    sem)

    def drain_out(l, slot):
        sem = osems[slot]
        for st, obuf in ((0, out_s), (1, out_a)):
            for dt in range(DT):
                pltpu.make_async_copy(
                    obuf.at[slot, pl.ds(dt * TILE, TILE)],
                    out_hbm.at[st, l, dt, pl.ds(wid * TILE, TILE)],
                    sem).wait()

    fire_gathers(0, 0)

    def step(i, _):
        l0 = i * 2
        for slot in range(2):
            l = l0 + slot
            nxt = 1 - slot

            @pl.when(l + 1 < LEN_CONTEXT)
            def _():
                fire_gathers(l + 1, nxt)

            drain_gathers(slot)

            @pl.when(l >= 2)
            def _():
                drain_out(l - 2, slot)

            compute(slot)
            fire_out(l, slot)
        return 0

    lax.fori_loop(0, LEN_CONTEXT // 2, step, 0)
    drain_out(LEN_CONTEXT - 2, 0)
    drain_out(LEN_CONTEXT - 1, 1)


def kernel(state, action, local_position, W_state, W_action, W_pos):
    s = state.T.astype(jnp.int32)
    a = action.T.astype(jnp.int32)
    p = local_position.T.astype(jnp.int32)
    out = _embed_sc(s, a, p, W_state, W_action, W_pos)
    out6 = out.reshape(2, LEN_CONTEXT, DT, NW, 8, BT)
    return out6.transpose(0, 3, 5, 1, 2, 4).reshape(2, BATCH, LEN_CONTEXT, D)


# speedup vs baseline: 1.4478x; 1.4478x over previous
import functools

import jax
import jax.numpy as jnp
from jax import lax
from jax.experimental import pallas as pl
from jax.experimental.pallas import tpu as pltpu
from jax.experimental.pallas import tpu_sc as plsc

D = 32
LANES = 16
NC, NS = 2, 16
NW = NC * NS

BATCH = 4096
LEN_CONTEXT = 200
BT = BATCH // NW
DT = D // 8
G = 2
NCHUNK = LEN_CONTEXT // G
PITCH = BT + 1

_mesh = plsc.VectorSubcoreMesh(core_axis_name="c", subcore_axis_name="s")


@functools.partial(
    pl.kernel,
    mesh=_mesh,
    compiler_params=pltpu.CompilerParams(
        use_tc_tiling_on_sc=False, needs_layout_passes=False),
    out_type=jax.ShapeDtypeStruct((2, LEN_CONTEXT, DT, NW, 8, BT), jnp.float32),
    scratch_types=[
        pltpu.VMEM((4, G, BT), jnp.int32),
        pltpu.VMEM((4, G, BT), jnp.int32),
        pltpu.VMEM((4, G, BT), jnp.int32),
        pltpu.VMEM((2, G * BT, D), jnp.float32),
        pltpu.VMEM((2, G * BT, D), jnp.float32),
        pltpu.VMEM((2, G * BT, D), jnp.float32),
        pltpu.VMEM((2, D, PITCH), jnp.float32),
        pltpu.VMEM((2, D, PITCH), jnp.float32),
        pltpu.SemaphoreType.DMA,
        pltpu.SemaphoreType.DMA,
        pltpu.SemaphoreType.DMA,
        pltpu.SemaphoreType.DMA,
        pltpu.SemaphoreType.DMA,
        pltpu.SemaphoreType.DMA,
    ],
)
def _embed_sc(state_hbm, action_hbm, lp_hbm, ws_hbm, wa_hbm, wp_hbm,
              out_hbm, idx_s, idx_a, idx_p, rows_s, rows_a, rows_p,
              out_s, out_a, isem0, isem1, gsem0, gsem1, osem0, osem1):
    wid = lax.axis_index("s") * NC + lax.axis_index("c")
    b0 = wid * BT
    diota = lax.iota(jnp.int32, LANES)

    isems = (isem0, isem1)
    gsems = (gsem0, gsem1)
    osems = (osem0, osem1)

    def idx_dma(c, s4, sem, wait):
        for src, dst in ((state_hbm, idx_s), (action_hbm, idx_a),
                         (lp_hbm, idx_p)):
            cp = pltpu.make_async_copy(
                src.at[pl.ds(c * G, G), pl.ds(b0, BT)], dst.at[s4], sem)
            if wait:
                cp.wait()
            else:
                cp.start()

    def gathers(s4, s2, wait):
        sem = gsems[s2]
        for tbl, idx, rows in ((ws_hbm, idx_s, rows_s),
                               (wa_hbm, idx_a, rows_a),
                               (wp_hbm, idx_p, rows_p)):
            for j in range(G):
                cp = pltpu.make_async_copy(
                    tbl.at[idx.at[s4, j]],
                    rows.at[s2, pl.ds(j * BT, BT)], sem)
                if wait:
                    cp.wait()
                else:
                    cp.start()

    def out_dma(l, lslot, wait):
        sem = osems[lslot]
        for st, obuf in ((0, out_s), (1, out_a)):
            for dt in range(DT):
                cp = pltpu.make_async_copy(
                    obuf.at[lslot, pl.ds(dt * 8, 8), pl.ds(0, BT)],
                    out_hbm.at[st, l, dt, wid], sem)
                if wait:
                    cp.wait()
                else:
                    cp.start()

    def compute(c, s2):
        for loff in range(G):
            l = c * G + loff
            out_dma(l, loff, wait=True)
            os_, oa = out_s.at[loff], out_a.at[loff]

            def body(r, _):
                bvec = jnp.broadcast_to(r, (LANES,))
                rr = loff * BT + r
                for h in range(2):
                    sl = pl.ds(h * LANES, LANES)
                    dvec = diota + h * LANES
                    p1 = rows_p[s2, rr, sl] + 1.0
                    for rows, obuf in ((rows_s, os_), (rows_a, oa)):
                        x = rows[s2, rr, sl]
                        e = jnp.exp(x + x)
                        rcp = 1.0 / (e + 1.0)
                        plsc.store_scatter(obuf, [dvec, bvec],
                                           p1 - (rcp + rcp))
                return 0

            lax.fori_loop(0, BT, body, 0)
            out_dma(l, loff, wait=False)

    out_dma(LEN_CONTEXT - 2, 0, wait=False)
    out_dma(LEN_CONTEXT - 1, 1, wait=False)
    idx_dma(0, 0, isems[0], wait=False)
    idx_dma(0, 0, isems[0], wait=True)
    idx_dma(1, 1, isems[1], wait=False)
    gathers(0, 0, wait=False)

    def step(i, _):
        for s in range(4):
            c = i * 4 + s
            s2 = s % 2

            @pl.when(c + 2 < NCHUNK)
            def _():
                idx_dma(c + 2, (s + 2) % 4, isems[s2], wait=False)

            @pl.when(c + 1 < NCHUNK)
            def _():
                idx_dma(c + 1, (s + 1) % 4, isems[1 - s2], wait=True)
                gathers((s + 1) % 4, 1 - s2, wait=False)

            gathers(s, s2, wait=True)
            compute(c, s2)
        return 0

    lax.fori_loop(0, NCHUNK // 4, step, 0)
    out_dma(LEN_CONTEXT - 2, 0, wait=True)
    out_dma(LEN_CONTEXT - 1, 1, wait=True)


def kernel(state, action, local_position, W_state, W_action, W_pos):
    s = state.T.astype(jnp.int32)
    a = action.T.astype(jnp.int32)
    p = local_position.T.astype(jnp.int32)
    out6 = _embed_sc(s, a, p, W_state, W_action, W_pos)
    return out6.transpose(0, 3, 5, 1, 2, 4).reshape(2, BATCH, LEN_CONTEXT, D)


# speedup vs baseline: 3.2102x; 2.2173x over previous
import functools

import jax
import jax.numpy as jnp
from jax import lax
from jax.experimental import pallas as pl
from jax.experimental.pallas import tpu as pltpu
from jax.experimental.pallas import tpu_sc as plsc

D = 32
LANES = 16
NC, NS = 2, 16
NW = NC * NS

BATCH = 4096
LEN_CONTEXT = 200
BT = BATCH // NW
DT = D // 8
G = 2
NCHUNK = LEN_CONTEXT // G
PITCH = BT + 1

_mesh = plsc.VectorSubcoreMesh(core_axis_name="c", subcore_axis_name="s")


@functools.partial(
    pl.kernel,
    mesh=_mesh,
    compiler_params=pltpu.CompilerParams(
        use_tc_tiling_on_sc=False, needs_layout_passes=False),
    out_type=jax.ShapeDtypeStruct((2, LEN_CONTEXT, DT, NW, 8, BT), jnp.float32),
    scratch_types=[
        pltpu.VMEM((4, G, BT), jnp.int32),
        pltpu.VMEM((4, G, BT), jnp.int32),
        pltpu.VMEM((4, G, BT), jnp.int32),
        pltpu.VMEM((2, G * BT, D), jnp.float32),
        pltpu.VMEM((2, G * BT, D), jnp.float32),
        pltpu.VMEM((2, G * BT, D), jnp.float32),
        pltpu.VMEM((2, D, PITCH), jnp.float32),
        pltpu.VMEM((2, D, PITCH), jnp.float32),
        pltpu.SemaphoreType.DMA,
        pltpu.SemaphoreType.DMA,
        pltpu.SemaphoreType.DMA,
        pltpu.SemaphoreType.DMA,
        pltpu.SemaphoreType.DMA,
        pltpu.SemaphoreType.DMA,
    ],
)
def _embed_sc(state_hbm, action_hbm, lp_hbm, ws_hbm, wa_hbm, wp_hbm,
              out_hbm, idx_s, idx_a, idx_p, rows_s, rows_a, rows_p,
              out_s, out_a, isem0, isem1, gsem0, gsem1, osem0, osem1):
    wid = lax.axis_index("s") * NC + lax.axis_index("c")
    b0 = wid * BT
    diota = lax.iota(jnp.int32, LANES)

    isems = (isem0, isem1)
    gsems = (gsem0, gsem1)
    osems = (osem0, osem1)

    def idx_dma(c, s4, sem, wait):
        for src, dst in ((state_hbm, idx_s), (action_hbm, idx_a),
                         (lp_hbm, idx_p)):
            cp = pltpu.make_async_copy(
                src.at[pl.ds(c * G, G), pl.ds(b0, BT)], dst.at[s4], sem)
            if wait:
                cp.wait()
            else:
                cp.start()

    def gathers(s4, s2, wait):
        sem = gsems[s2]
        for tbl, idx, rows in ((ws_hbm, idx_s, rows_s),
                               (wa_hbm, idx_a, rows_a),
                               (wp_hbm, idx_p, rows_p)):
            for j in range(G):
                cp = pltpu.make_async_copy(
                    tbl.at[idx.at[s4, j]],
                    rows.at[s2, pl.ds(j * BT, BT)], sem)
                if wait:
                    cp.wait()
                else:
                    cp.start()

    def out_dma(l, lslot, wait):
        sem = osems[lslot]
        for st, obuf in ((0, out_s), (1, out_a)):
            for dt in range(DT):
                cp = pltpu.make_async_copy(
                    obuf.at[lslot, pl.ds(dt * 8, 8), pl.ds(0, BT)],
                    out_hbm.at[st, l, dt, wid], sem)
                if wait:
                    cp.wait()
                else:
                    cp.start()

    def compute(c, s2):
        for loff in range(G):
            l = c * G + loff
            out_dma(l, loff, wait=True)
            os_, oa = out_s.at[loff], out_a.at[loff]

            @plsc.parallel_loop(0, BT, step=1, unroll=2)
            def _(r):
                bvec = jnp.broadcast_to(r, (LANES,))
                rr = loff * BT + r
                for h in range(2):
                    sl = pl.ds(h * LANES, LANES)
                    dvec = diota + h * LANES
                    p1 = rows_p[s2, rr, sl] + 1.0
                    for rows, obuf in ((rows_s, os_), (rows_a, oa)):
                        x = rows[s2, rr, sl]
                        e = jnp.exp(x + x)
                        rcp = 1.0 / (e + 1.0)
                        plsc.store_scatter(obuf, [dvec, bvec],
                                           p1 - (rcp + rcp))
            out_dma(l, loff, wait=False)

    out_dma(LEN_CONTEXT - 2, 0, wait=False)
    out_dma(LEN_CONTEXT - 1, 1, wait=False)
    idx_dma(0, 0, isems[0], wait=False)
    idx_dma(0, 0, isems[0], wait=True)
    idx_dma(1, 1, isems[1], wait=False)
    gathers(0, 0, wait=False)

    def step(i, _):
        for s in range(4):
            c = i * 4 + s
            s2 = s % 2

            @pl.when(c + 2 < NCHUNK)
            def _():
                idx_dma(c + 2, (s + 2) % 4, isems[s2], wait=False)

            @pl.when(c + 1 < NCHUNK)
            def _():
                idx_dma(c + 1, (s + 1) % 4, isems[1 - s2], wait=True)
                gathers((s + 1) % 4, 1 - s2, wait=False)

            gathers(s, s2, wait=True)
            compute(c, s2)
        return 0

    lax.fori_loop(0, NCHUNK // 4, step, 0)
    out_dma(LEN_CONTEXT - 2, 0, wait=True)
    out_dma(LEN_CONTEXT - 1, 1, wait=True)


def kernel(state, action, local_position, W_state, W_action, W_pos):
    s = state.T.astype(jnp.int32)
    a = action.T.astype(jnp.int32)
    p = local_position.T.astype(jnp.int32)
    out6 = _embed_sc(s, a, p, W_state, W_action, W_pos)
    return out6.transpose(0, 3, 5, 1, 2, 4).reshape(2, BATCH, LEN_CONTEXT, D)
